# baseline (speedup 1.0000x reference)
import functools

import jax
import jax.numpy as jnp
from jax import lax
from jax.experimental import pallas as pl
from jax.experimental.pallas import tpu as pltpu
from jax.experimental.pallas import tpu_sc as plsc

N = 50000
NNEI = 64
RCUT = 6.0
RCUT_SMTH = 0.5

NW = 32
APT = 1568
CHUNKS = 4
APC = APT // CHUNKS
EPC = APC * NNEI
EDGES = N * NNEI

BLK = 1000


def _edge_vreg(table_v, idx_v, acc_v, cv, e0, comp):
    iv = idx_v[pl.ds(e0, 16)]
    nv = plsc.load_gather(table_v, [iv])
    d = nv - cv
    d2 = d * d
    if comp == 0:
        acc_v[pl.ds(e0, 16)] = d2
    elif comp == 1:
        acc_v[pl.ds(e0, 16)] += d2
    else:
        r2 = acc_v[pl.ds(e0, 16)] + d2 + 1e-12
        seed = jnp.int32(0x5F3759DF) - (plsc.bitcast(r2, jnp.int32) >> 1)
        y = plsc.bitcast(seed, jnp.float32)
        hr = 0.5 * r2
        for _ in range(3):
            y = y * (1.5 - hr * y * y)
        r = r2 * y
        uu = (r - RCUT_SMTH) * (1.0 / (RCUT - RCUT_SMTH))
        vv = uu * uu * uu * (uu * (-6.0 * uu + 15.0) - 10.0) + 1.0
        sw = jnp.where(r < RCUT_SMTH, 1.0, jnp.where(r > RCUT, 0.0, vv))
        acc_v[pl.ds(e0, 16)] = sw * y


def _sc_body(cx_hbm, cy_hbm, cz_hbm, nlist_hbm, sr_hbm, table_v, idx_v, acc_v):
    wid = lax.axis_index("s") * 2 + lax.axis_index("c")
    comp_hbms = (cx_hbm, cy_hbm, cz_hbm)

    for c in range(CHUNKS):
        edge_base = jnp.minimum(wid * (APT * NNEI) + c * EPC, EDGES - EPC)
        atom_base = edge_base // NNEI
        pltpu.sync_copy(nlist_hbm.at[pl.ds(edge_base, EPC)], idx_v)

        for comp in range(3):
            pltpu.sync_copy(comp_hbms[comp], table_v)

            def atom_body(t, _, comp=comp, atom_base=atom_base):
                for u in range(2):
                    a = t * 2 + u
                    cv = plsc.load_gather(
                        table_v, [jnp.full((16,), atom_base + a, jnp.int32)])
                    for q in range(4):
                        _edge_vreg(table_v, idx_v, acc_v, cv,
                                   a * NNEI + q * 16, comp)
                return _

            lax.fori_loop(0, APC // 2, atom_body, None)

        pltpu.sync_copy(acc_v, sr_hbm.at[pl.ds(edge_base, EPC)])


@functools.partial(
    pl.kernel,
    out_type=jax.ShapeDtypeStruct((EDGES,), jnp.float32),
    mesh=plsc.VectorSubcoreMesh(core_axis_name="c", subcore_axis_name="s",
                                num_cores=2, num_subcores=16),
    compiler_params=pltpu.CompilerParams(use_tc_tiling_on_sc=False,
                                         needs_layout_passes=False),
    scratch_types=[
        pltpu.VMEM((N,), jnp.float32),
        pltpu.VMEM((EPC,), jnp.int32),
        pltpu.VMEM((EPC,), jnp.float32),
    ],
)
def _sc_descriptor(cx_hbm, cy_hbm, cz_hbm, nlist_hbm, sr_hbm,
                   table_v, idx_v, acc_v):
    _sc_body(cx_hbm, cy_hbm, cz_hbm, nlist_hbm, sr_hbm, table_v, idx_v, acc_v)


def _head_body(sr_ref,
               W0_ref, b0_ref, W1_ref, b1_ref,
               Wf0x_ref, bf0x_ref, Wf1x_ref, bf1x_ref, dt1x_ref, Woutx_ref,
               bout_ref, eraw_ref, esum_ref, dbuf_ref):
    def srow(s, _):
        srs = sr_ref[pl.ds(s * 8, 8), :]
        g1 = [jnp.tanh(srs * W0_ref[0, j] + b0_ref[0, j]) for j in range(8)]
        cols = []
        for k in range(16):
            z = g1[0] * W1_ref[0, k]
            for j in range(1, 8):
                z = z + g1[j] * W1_ref[j, k]
            g2k = jnp.tanh(z + b1_ref[0, k])
            cols.append(jnp.sum(g2k[:, :NNEI], axis=1, keepdims=True)
                        * (1.0 / NNEI))
            cols.append(jnp.sum(g2k[:, NNEI:], axis=1, keepdims=True)
                        * (1.0 / NNEI))
        dbuf_ref[pl.ds(s * 8, 8), :] = jnp.concatenate(cols, axis=1)
        return _

    lax.fori_loop(0, BLK // 8, srow, None)
    D = dbuf_ref[...]

    h = jnp.tanh(jnp.dot(D, Wf0x_ref[...], preferred_element_type=jnp.float32)
                 + bf0x_ref[...])
    h = h + jnp.tanh(jnp.dot(h, Wf1x_ref[...], preferred_element_type=jnp.float32)
                     + bf1x_ref[...]) * dt1x_ref[...]
    ae = (jnp.dot(h, Woutx_ref[...], preferred_element_type=jnp.float32)
          + bout_ref[0, 0])
    eraw_ref[...] = ae

    @pl.when(pl.program_id(0) == 0)
    def _():
        esum_ref[...] = jnp.zeros_like(esum_ref)

    esum_ref[...] += jnp.sum(ae, keepdims=True)


def _expand2(W):
    K, M = W.shape
    z = jnp.zeros((2 * K, 2 * M), jnp.float32)
    return z.at[0::2, 0::2].set(W).at[1::2, 1::2].set(W)


def _head(sr2, W0, b0, W1, b1, Wf0, bf0, Wf1, bf1, dt1, Wout, bout):
    grid = (N // 2) // BLK
    smem = functools.partial(pl.BlockSpec, memory_space=pltpu.SMEM)
    full = lambda shape: pl.BlockSpec(shape, lambda i: (0, 0))
    eraw2, esum = pl.pallas_call(
        _head_body,
        grid=(grid,),
        in_specs=[
            pl.BlockSpec((BLK, 2 * NNEI), lambda i: (i, 0)),
            smem((1, 8), lambda i: (0, 0)),
            smem((1, 8), lambda i: (0, 0)),
            smem((8, 16), lambda i: (0, 0)),
            smem((1, 16), lambda i: (0, 0)),
            full((32, 64)),
            full((1, 64)),
            full((64, 64)),
            full((1, 64)),
            full((1, 64)),
            full((64, 2)),
            smem((1, 1), lambda i: (0, 0)),
        ],
        out_specs=[
            pl.BlockSpec((BLK, 2), lambda i: (i, 0)),
            pl.BlockSpec((1, 1), lambda i: (0, 0)),
        ],
        out_shape=[
            jax.ShapeDtypeStruct((N // 2, 2), jnp.float32),
            jax.ShapeDtypeStruct((1, 1), jnp.float32),
        ],
        scratch_shapes=[pltpu.VMEM((BLK, 32), jnp.float32)],
    )(sr2,
      W0, b0.reshape(1, 8), W1, b1.reshape(1, 16),
      _expand2(Wf0), jnp.repeat(bf0, 2).reshape(1, 64),
      _expand2(Wf1), jnp.repeat(bf1, 2).reshape(1, 64),
      jnp.repeat(dt1, 2).reshape(1, 64), _expand2(Wout),
      bout.reshape(1, 1))
    return esum, eraw2


def kernel(coord, nlist, W0, b0, W1, b1, Wf0, bf0, Wf1, bf1, dt1, Wout, bout):
    ct = coord.T
    sr2 = _sc_descriptor(ct[0], ct[1], ct[2],
                         nlist.reshape(EDGES)).reshape(N // 2, 2 * NNEI)
    esum, eraw2 = _head(sr2, W0, b0, W1, b1,
                        Wf0, bf0, Wf1, bf1, dt1, Wout, bout)
    return esum.reshape(1), eraw2.reshape(1, N)


# speedup vs baseline: 2.9940x; 2.9940x over previous
import functools

import jax
import jax.numpy as jnp
from jax import lax
from jax.experimental import pallas as pl
from jax.experimental.pallas import tpu as pltpu
from jax.experimental.pallas import tpu_sc as plsc

N = 50000
NNEI = 64
RCUT = 6.0
RCUT_SMTH = 0.5

NW = 32
APT = 1568
CHUNKS = 8
APC = APT // CHUNKS
EPC = APC * NNEI
EDGES = N * NNEI

BLK = 1000


def _edge_vreg(table_v, idx_v, acc_v, cv, e0, comp):
    iv = idx_v[pl.ds(e0, 16)]
    nv = plsc.load_gather(table_v, [iv])
    d = nv - cv
    d2 = d * d
    if comp == 0:
        acc_v[pl.ds(e0, 16)] = d2
    elif comp == 1:
        acc_v[pl.ds(e0, 16)] += d2
    else:
        r2 = acc_v[pl.ds(e0, 16)] + d2 + 1e-12
        seed = jnp.int32(0x5F3759DF) - (plsc.bitcast(r2, jnp.int32) >> 1)
        y = plsc.bitcast(seed, jnp.float32)
        hr = 0.5 * r2
        for _ in range(3):
            y = y * (1.5 - hr * y * y)
        r = r2 * y
        uu = (r - RCUT_SMTH) * (1.0 / (RCUT - RCUT_SMTH))
        vv = uu * uu * uu * (uu * (-6.0 * uu + 15.0) - 10.0) + 1.0
        sw = jnp.where(r < RCUT_SMTH, 1.0, jnp.where(r > RCUT, 0.0, vv))
        acc_v[pl.ds(e0, 16)] = sw * y


def _sc_body(cx_hbm, cy_hbm, cz_hbm, nlist_hbm, sr_hbm,
             table0_v, table1_v, idx_v, acc_v, sem0, sem1):
    wid = lax.axis_index("s") * 2 + lax.axis_index("c")
    comp_hbms = (cx_hbm, cy_hbm, cz_hbm)
    bufs = (table0_v, table1_v)
    sems = (sem0, sem1)
    passes = [(c, comp) for c in range(CHUNKS) for comp in range(3)]

    pltpu.async_copy(comp_hbms[0], bufs[0], sems[0])
    for i, (c, comp) in enumerate(passes):
        if i + 1 < len(passes):
            nxt = passes[i + 1][1]
            pltpu.async_copy(comp_hbms[nxt], bufs[(i + 1) % 2],
                             sems[(i + 1) % 2])
        edge_base = jnp.minimum(wid * (APT * NNEI) + c * EPC, EDGES - EPC)
        atom_base = edge_base // NNEI
        if comp == 0:
            pltpu.sync_copy(nlist_hbm.at[pl.ds(edge_base, EPC)], idx_v)
        table_v = bufs[i % 2]
        pltpu.make_async_copy(comp_hbms[comp], table_v, sems[i % 2]).wait()

        def atom_body(t, _, comp=comp, atom_base=atom_base, table_v=table_v):
            for u in range(2):
                a = t * 2 + u
                cv = plsc.load_gather(
                    table_v, [jnp.full((16,), atom_base + a, jnp.int32)])
                for q in range(4):
                    _edge_vreg(table_v, idx_v, acc_v, cv,
                               a * NNEI + q * 16, comp)
            return _

        lax.fori_loop(0, APC // 2, atom_body, None)
        if comp == 2:
            pltpu.sync_copy(acc_v, sr_hbm.at[pl.ds(edge_base, EPC)])


@functools.partial(
    pl.kernel,
    out_type=jax.ShapeDtypeStruct((EDGES,), jnp.float32),
    mesh=plsc.VectorSubcoreMesh(core_axis_name="c", subcore_axis_name="s",
                                num_cores=2, num_subcores=16),
    compiler_params=pltpu.CompilerParams(use_tc_tiling_on_sc=False,
                                         needs_layout_passes=False),
    scratch_types=[
        pltpu.VMEM((N,), jnp.float32),
        pltpu.VMEM((N,), jnp.float32),
        pltpu.VMEM((EPC,), jnp.int32),
        pltpu.VMEM((EPC,), jnp.float32),
        pltpu.SemaphoreType.DMA,
        pltpu.SemaphoreType.DMA,
    ],
)
def _sc_descriptor(cx_hbm, cy_hbm, cz_hbm, nlist_hbm, sr_hbm,
                   table0_v, table1_v, idx_v, acc_v, sem0, sem1):
    _sc_body(cx_hbm, cy_hbm, cz_hbm, nlist_hbm, sr_hbm,
             table0_v, table1_v, idx_v, acc_v, sem0, sem1)


def _head_body(sr_ref,
               W0_ref, b0_ref, W1_ref, b1_ref,
               Wf0x_ref, bf0x_ref, Wf1x_ref, bf1x_ref, dt1x_ref, Woutx_ref,
               bout_ref, eraw_ref, esum_ref, dbuf_ref):
    sr = sr_ref[...]
    del dbuf_ref
    g1 = [jnp.tanh(sr * W0_ref[0, j] + b0_ref[0, j]) for j in range(8)]
    cols = []
    for k in range(16):
        z = g1[0] * W1_ref[0, k]
        for j in range(1, 8):
            z = z + g1[j] * W1_ref[j, k]
        g2k = jnp.tanh(z + b1_ref[0, k])
        cols.append(jnp.sum(g2k[:, :NNEI], axis=1, keepdims=True) * (1.0 / NNEI))
        cols.append(jnp.sum(g2k[:, NNEI:], axis=1, keepdims=True) * (1.0 / NNEI))
    D = jnp.concatenate(cols, axis=1)

    h = jnp.tanh(jnp.dot(D, Wf0x_ref[...], preferred_element_type=jnp.float32)
                 + bf0x_ref[...])
    h = h + jnp.tanh(jnp.dot(h, Wf1x_ref[...], preferred_element_type=jnp.float32)
                     + bf1x_ref[...]) * dt1x_ref[...]
    ae = (jnp.dot(h, Woutx_ref[...], preferred_element_type=jnp.float32)
          + bout_ref[0, 0])
    eraw_ref[...] = ae

    @pl.when(pl.program_id(0) == 0)
    def _():
        esum_ref[...] = jnp.zeros_like(esum_ref)

    esum_ref[...] += jnp.sum(ae, keepdims=True)


def _expand2(W):
    K, M = W.shape
    z = jnp.zeros((2 * K, 2 * M), jnp.float32)
    return z.at[0::2, 0::2].set(W).at[1::2, 1::2].set(W)


def _head(sr2, W0, b0, W1, b1, Wf0, bf0, Wf1, bf1, dt1, Wout, bout):
    grid = (N // 2) // BLK
    smem = functools.partial(pl.BlockSpec, memory_space=pltpu.SMEM)
    full = lambda shape: pl.BlockSpec(shape, lambda i: (0, 0))
    eraw2, esum = pl.pallas_call(
        _head_body,
        grid=(grid,),
        in_specs=[
            pl.BlockSpec((BLK, 2 * NNEI), lambda i: (i, 0)),
            smem((1, 8), lambda i: (0, 0)),
            smem((1, 8), lambda i: (0, 0)),
            smem((8, 16), lambda i: (0, 0)),
            smem((1, 16), lambda i: (0, 0)),
            full((32, 64)),
            full((1, 64)),
            full((64, 64)),
            full((1, 64)),
            full((1, 64)),
            full((64, 2)),
            smem((1, 1), lambda i: (0, 0)),
        ],
        out_specs=[
            pl.BlockSpec((BLK, 2), lambda i: (i, 0)),
            pl.BlockSpec((1, 1), lambda i: (0, 0)),
        ],
        out_shape=[
            jax.ShapeDtypeStruct((N // 2, 2), jnp.float32),
            jax.ShapeDtypeStruct((1, 1), jnp.float32),
        ],
        scratch_shapes=[pltpu.VMEM((BLK, 32), jnp.float32)],
    )(sr2,
      W0, b0.reshape(1, 8), W1, b1.reshape(1, 16),
      _expand2(Wf0), jnp.repeat(bf0, 2).reshape(1, 64),
      _expand2(Wf1), jnp.repeat(bf1, 2).reshape(1, 64),
      jnp.repeat(dt1, 2).reshape(1, 64), _expand2(Wout),
      bout.reshape(1, 1))
    return esum, eraw2


def kernel(coord, nlist, W0, b0, W1, b1, Wf0, bf0, Wf1, bf1, dt1, Wout, bout):
    ct = coord.T
    sr2 = _sc_descriptor(ct[0], ct[1], ct[2],
                         nlist.reshape(EDGES)).reshape(N // 2, 2 * NNEI)
    esum, eraw2 = _head(sr2, W0, b0, W1, b1,
                        Wf0, bf0, Wf1, bf1, dt1, Wout, bout)
    return esum.reshape(1), eraw2.reshape(1, N)
